# baseline (speedup 1.0000x reference)
import functools

import jax
import jax.numpy as jnp
from jax import lax
from jax.experimental import pallas as pl
from jax.experimental.pallas import tpu as pltpu
from jax.experimental.pallas import tpu_sc as plsc

CHUNK = 64
WREP = 128


def _cumsum_shift(x, axis, n):
    s = 1
    while s < n:
        if axis == 0:
            shifted = jnp.concatenate(
                [jnp.zeros((s, x.shape[1]), x.dtype), x[:-s, :]], axis=0)
        else:
            shifted = jnp.concatenate(
                [jnp.zeros((x.shape[0], s), x.dtype), x[:, :-s]], axis=1)
        x = x + shifted
        s *= 2
    return x


def _router_body(x_ref, wr_ref, pos_ref, w16_ref, cnt_ref, off_ref, *, E, EP):
    x = x_ref[...]
    T = x.shape[0]
    wr = jnp.concatenate(
        [wr_ref[...], jnp.zeros((x.shape[1], EP - E), jnp.float32)], axis=1)
    logits = jnp.dot(x, wr, preferred_element_type=jnp.float32)
    lane = jax.lax.broadcasted_iota(jnp.int32, logits.shape, 1)
    logits = jnp.where(lane < E, logits, -1e30)
    m = jnp.max(logits, axis=-1, keepdims=True)
    p = jnp.exp(logits - m)
    p = p / jnp.sum(p, axis=-1, keepdims=True)
    pmax = jnp.max(p, axis=-1, keepdims=True)
    e_idx = jnp.min(jnp.where(p == pmax, lane, EP), axis=-1, keepdims=True)
    onehot = (lane == e_idx).astype(jnp.float32)
    counts = jnp.sum(onehot, axis=0, keepdims=True)
    cpad = jnp.floor((counts + 7.0) / 8.0) * 8.0
    off_excl = _cumsum_shift(cpad, 1, EP) - cpad
    rank = _cumsum_shift(onehot, 0, T) - onehot
    pos = jnp.sum(onehot * (rank + off_excl), axis=-1, keepdims=True)
    pos_ref[...] = jnp.broadcast_to(pos, (T, EP)).astype(jnp.int32)
    w16_ref[...] = jnp.broadcast_to(pmax, (T, WREP))
    cnt_ref[...] = counts.astype(jnp.int32)
    off_ref[...] = off_excl.astype(jnp.int32)


def _ffn_body(xs_ref, ws_ref, wg_ref, wu_ref, wd_ref, off_ref, cnt_ref,
              y_ref, *, MAXCH):
    e = pl.program_id(0)
    f = pl.program_id(1)
    off = off_ref[0, e]
    cnt = cnt_ref[0, e]
    wg = wg_ref[0]
    wu = wu_ref[0]
    wd = wd_ref[0]

    def chunk(i, _):
        @pl.when(i * CHUNK < cnt)
        def _do():
            start = pl.multiple_of(off + i * CHUNK, 8)
            rows = xs_ref[pl.ds(start, CHUNK), :]
            gate = jnp.dot(rows, wg, preferred_element_type=jnp.float32)
            up = jnp.dot(rows, wu, preferred_element_type=jnp.float32)
            act = up * (gate * jax.nn.sigmoid(gate))
            y = jnp.dot(act, wd, preferred_element_type=jnp.float32)
            y = y * ws_ref[pl.ds(start, CHUNK), 0:1]

            @pl.when(f == 0)
            def _set():
                y_ref[pl.ds(start, CHUNK), :] = y

            @pl.when(f != 0)
            def _acc():
                y_ref[pl.ds(start, CHUNK), :] += y
        return 0

    jax.lax.fori_loop(0, MAXCH, chunk, 0)


def _make_sc_scatter(T, H, TPAD, NC, BW):
    mesh = plsc.VectorSubcoreMesh(core_axis_name="c", subcore_axis_name="s")

    @functools.partial(
        pl.kernel, mesh=mesh,
        out_type=[
            jax.ShapeDtypeStruct((TPAD, H), jnp.float32),
            jax.ShapeDtypeStruct((TPAD, WREP), jnp.float32),
        ],
        scratch_types=[
            pltpu.VMEM((BW,), jnp.int32),
            pltpu.VMEM((BW, H), jnp.float32),
            pltpu.VMEM((BW, WREP), jnp.float32),
            pltpu.SemaphoreType.DMA,
            pltpu.SemaphoreType.DMA,
        ],
    )
    def scatter_k(x_hbm, w16_hbm, pos_hbm, xs_hbm, ws_hbm,
                  idx_v, rows_v, wrow_v, sem1, sem2):
        wid = lax.axis_index("s") * NC + lax.axis_index("c")
        base = wid * BW
        pltpu.sync_copy(pos_hbm.at[pl.ds(base, BW)], idx_v)
        pltpu.sync_copy(x_hbm.at[pl.ds(base, BW)], rows_v)
        pltpu.sync_copy(w16_hbm.at[pl.ds(base, BW)], wrow_v)
        c1 = pltpu.async_copy(rows_v, xs_hbm.at[idx_v], sem1)
        c2 = pltpu.async_copy(wrow_v, ws_hbm.at[idx_v], sem2)
        c1.wait()
        c2.wait()

    return scatter_k


def _make_sc_gather(T, H, TPAD, NC, BW):
    mesh = plsc.VectorSubcoreMesh(core_axis_name="c", subcore_axis_name="s")

    @functools.partial(
        pl.kernel, mesh=mesh,
        out_type=jax.ShapeDtypeStruct((T, H), jnp.float32),
        scratch_types=[
            pltpu.VMEM((BW,), jnp.int32),
            pltpu.VMEM((BW, H), jnp.float32),
            pltpu.SemaphoreType.DMA,
        ],
    )
    def gather_k(ys_hbm, pos_hbm, out_hbm, idx_v, rows_v, sem):
        wid = lax.axis_index("s") * NC + lax.axis_index("c")
        base = wid * BW
        pltpu.sync_copy(pos_hbm.at[pl.ds(base, BW)], idx_v)
        pltpu.async_copy(ys_hbm.at[idx_v], rows_v, sem).wait()
        pltpu.sync_copy(rows_v, out_hbm.at[pl.ds(base, BW)])

    return gather_k


def kernel(hidden_states, Wr, Wg, Wu, Wd):
    b, s, h = hidden_states.shape
    T = b * s
    E, H, F = Wg.shape
    EP = 128
    flat = hidden_states.reshape(T, h)

    pos_b, w16, cnt2d, off2d = pl.pallas_call(
        functools.partial(_router_body, E=E, EP=EP),
        out_shape=[
            jax.ShapeDtypeStruct((T, EP), jnp.int32),
            jax.ShapeDtypeStruct((T, WREP), jnp.float32),
            jax.ShapeDtypeStruct((1, EP), jnp.int32),
            jax.ShapeDtypeStruct((1, EP), jnp.int32),
        ],
    )(flat, Wr)

    pos_1d = pos_b[:, 0]

    tpad = T + E * 7 + CHUNK
    TPAD = ((tpad + 127) // 128) * 128

    info = plsc.get_sparse_core_info()
    NC, NS = info.num_cores, info.num_subcores
    BW = T // (NC * NS)

    xs, ws16 = _make_sc_scatter(T, H, TPAD, NC, BW)(flat, w16, pos_1d)

    MAXCH = (T + CHUNK - 1) // CHUNK
    NF = 2
    FH = F // NF
    ys = pl.pallas_call(
        functools.partial(_ffn_body, MAXCH=MAXCH),
        grid=(E, NF),
        in_specs=[
            pl.BlockSpec((TPAD, H), lambda e, f: (0, 0)),
            pl.BlockSpec((TPAD, WREP), lambda e, f: (0, 0)),
            pl.BlockSpec((1, H, FH), lambda e, f: (e, 0, f)),
            pl.BlockSpec((1, H, FH), lambda e, f: (e, 0, f)),
            pl.BlockSpec((1, FH, H), lambda e, f: (e, f, 0)),
            pl.BlockSpec(memory_space=pltpu.SMEM),
            pl.BlockSpec(memory_space=pltpu.SMEM),
        ],
        out_specs=pl.BlockSpec((TPAD, H), lambda e, f: (0, 0)),
        out_shape=jax.ShapeDtypeStruct((TPAD, H), jnp.float32),
        compiler_params=pltpu.CompilerParams(
            dimension_semantics=("arbitrary", "arbitrary")),
    )(xs, ws16, Wg, Wu, Wd, off2d, cnt2d)

    out = _make_sc_gather(T, H, TPAD, NC, BW)(ys, pos_1d)

    return out.reshape(b, s, h)


# speedup vs baseline: 1.1305x; 1.1305x over previous
import functools

import jax
import jax.numpy as jnp
from jax import lax
from jax.experimental import pallas as pl
from jax.experimental.pallas import tpu as pltpu
from jax.experimental.pallas import tpu_sc as plsc

CHUNK = 64
WREP = 128


def _cumsum_shift(x, axis, n):
    s = 1
    while s < n:
        if axis == 0:
            shifted = jnp.concatenate(
                [jnp.zeros((s, x.shape[1]), x.dtype), x[:-s, :]], axis=0)
        else:
            shifted = jnp.concatenate(
                [jnp.zeros((x.shape[0], s), x.dtype), x[:, :-s]], axis=1)
        x = x + shifted
        s *= 2
    return x


def _router_body(x_ref, wr_ref, pos_ref, w16_ref, cnt_ref, off_ref, *, E, EP):
    x = x_ref[...]
    T = x.shape[0]
    wr = jnp.concatenate(
        [wr_ref[...], jnp.zeros((x.shape[1], EP - E), jnp.float32)], axis=1)
    logits = jnp.dot(x, wr, preferred_element_type=jnp.float32)
    lane = jax.lax.broadcasted_iota(jnp.int32, logits.shape, 1)
    logits = jnp.where(lane < E, logits, -1e30)
    m = jnp.max(logits, axis=-1, keepdims=True)
    p = jnp.exp(logits - m)
    p = p / jnp.sum(p, axis=-1, keepdims=True)
    pmax = jnp.max(p, axis=-1, keepdims=True)
    e_idx = jnp.min(jnp.where(p == pmax, lane, EP), axis=-1, keepdims=True)
    onehot = (lane == e_idx).astype(jnp.float32)
    counts = jnp.sum(onehot, axis=0, keepdims=True)
    cpad = jnp.floor((counts + 7.0) / 8.0) * 8.0
    off_excl = _cumsum_shift(cpad, 1, EP) - cpad
    rank = _cumsum_shift(onehot, 0, T) - onehot
    pos = jnp.sum(onehot * (rank + off_excl), axis=-1, keepdims=True)
    pos_ref[...] = jnp.broadcast_to(pos, (T, EP)).astype(jnp.int32)
    w16_ref[...] = jnp.broadcast_to(pmax, (T, WREP))
    cnt_ref[...] = counts.astype(jnp.int32)
    off_ref[...] = off_excl.astype(jnp.int32)


def _ffn_body(xs_ref, ws_ref, wg_ref, wu_ref, wd_ref, off_ref, cnt_ref,
              y_ref, *, MAXCH):
    e = pl.program_id(0)
    off = off_ref[0, e]
    cnt = cnt_ref[0, e]
    wg = wg_ref[0]
    wu = wu_ref[0]
    wd = wd_ref[0]

    def chunk(i, _):
        @pl.when(i * CHUNK < cnt)
        def _do():
            start = pl.multiple_of(off + i * CHUNK, 8)
            rows = xs_ref[pl.ds(start, CHUNK), :]
            gate = jnp.dot(rows, wg, preferred_element_type=jnp.float32)
            up = jnp.dot(rows, wu, preferred_element_type=jnp.float32)
            act = up * (gate * jax.nn.sigmoid(gate))
            y = jnp.dot(act, wd, preferred_element_type=jnp.float32)
            y_ref[pl.ds(start, CHUNK), :] = y * ws_ref[pl.ds(start, CHUNK), 0:1]
        return 0

    jax.lax.fori_loop(0, MAXCH, chunk, 0)


def _make_sc_scatter(T, H, TPAD, NC, BW):
    mesh = plsc.VectorSubcoreMesh(core_axis_name="c", subcore_axis_name="s")

    @functools.partial(
        pl.kernel, mesh=mesh,
        out_type=[
            jax.ShapeDtypeStruct((TPAD, H), jnp.float32),
            jax.ShapeDtypeStruct((TPAD, WREP), jnp.float32),
        ],
        scratch_types=[
            pltpu.VMEM((BW,), jnp.int32),
            pltpu.VMEM((BW, H), jnp.float32),
            pltpu.VMEM((BW, WREP), jnp.float32),
            pltpu.SemaphoreType.DMA,
            pltpu.SemaphoreType.DMA,
        ],
    )
    def scatter_k(x_hbm, w16_hbm, pos_hbm, xs_hbm, ws_hbm,
                  idx_v, rows_v, wrow_v, sem1, sem2):
        wid = lax.axis_index("s") * NC + lax.axis_index("c")
        base = wid * BW
        pltpu.sync_copy(pos_hbm.at[pl.ds(base, BW)], idx_v)
        pltpu.sync_copy(x_hbm.at[pl.ds(base, BW)], rows_v)
        pltpu.sync_copy(w16_hbm.at[pl.ds(base, BW)], wrow_v)
        c1 = pltpu.async_copy(rows_v, xs_hbm.at[idx_v], sem1)
        c2 = pltpu.async_copy(wrow_v, ws_hbm.at[idx_v], sem2)
        c1.wait()
        c2.wait()

    return scatter_k


def _make_sc_gather(T, H, TPAD, NC, BW):
    mesh = plsc.VectorSubcoreMesh(core_axis_name="c", subcore_axis_name="s")

    @functools.partial(
        pl.kernel, mesh=mesh,
        out_type=jax.ShapeDtypeStruct((T, H), jnp.float32),
        scratch_types=[
            pltpu.VMEM((BW,), jnp.int32),
            pltpu.VMEM((BW, H), jnp.float32),
            pltpu.SemaphoreType.DMA,
        ],
    )
    def gather_k(ys_hbm, pos_hbm, out_hbm, idx_v, rows_v, sem):
        wid = lax.axis_index("s") * NC + lax.axis_index("c")
        base = wid * BW
        pltpu.sync_copy(pos_hbm.at[pl.ds(base, BW)], idx_v)
        pltpu.async_copy(ys_hbm.at[idx_v], rows_v, sem).wait()
        pltpu.sync_copy(rows_v, out_hbm.at[pl.ds(base, BW)])

    return gather_k


def kernel(hidden_states, Wr, Wg, Wu, Wd):
    b, s, h = hidden_states.shape
    T = b * s
    E, H, F = Wg.shape
    EP = 128
    flat = hidden_states.reshape(T, h)

    pos_b, w16, cnt2d, off2d = pl.pallas_call(
        functools.partial(_router_body, E=E, EP=EP),
        out_shape=[
            jax.ShapeDtypeStruct((T, EP), jnp.int32),
            jax.ShapeDtypeStruct((T, WREP), jnp.float32),
            jax.ShapeDtypeStruct((1, EP), jnp.int32),
            jax.ShapeDtypeStruct((1, EP), jnp.int32),
        ],
    )(flat, Wr)

    pos_1d = pos_b[:, 0]

    tpad = T + E * 7 + CHUNK
    TPAD = ((tpad + 127) // 128) * 128

    info = plsc.get_sparse_core_info()
    NC, NS = info.num_cores, info.num_subcores
    BW = T // (NC * NS)

    xs, ws16 = _make_sc_scatter(T, H, TPAD, NC, BW)(flat, w16, pos_1d)

    MAXCH = (T + CHUNK - 1) // CHUNK
    ys = pl.pallas_call(
        functools.partial(_ffn_body, MAXCH=MAXCH),
        grid=(E,),
        in_specs=[
            pl.BlockSpec((TPAD, H), lambda e: (0, 0)),
            pl.BlockSpec((TPAD, WREP), lambda e: (0, 0)),
            pl.BlockSpec((1, H, F), lambda e: (e, 0, 0)),
            pl.BlockSpec((1, H, F), lambda e: (e, 0, 0)),
            pl.BlockSpec((1, F, H), lambda e: (e, 0, 0)),
            pl.BlockSpec(memory_space=pltpu.SMEM),
            pl.BlockSpec(memory_space=pltpu.SMEM),
        ],
        out_specs=pl.BlockSpec((TPAD, H), lambda e: (0, 0)),
        out_shape=jax.ShapeDtypeStruct((TPAD, H), jnp.float32),
        compiler_params=pltpu.CompilerParams(
            dimension_semantics=("arbitrary",)),
    )(xs, ws16, Wg, Wu, Wd, off2d, cnt2d)

    out = _make_sc_gather(T, H, TPAD, NC, BW)(ys, pos_1d)

    return out.reshape(b, s, h)
